# baseline (speedup 1.0000x reference)
import functools
import math

import jax
import jax.numpy as jnp
from jax import lax
from jax.experimental import pallas as pl
from jax.experimental.pallas import tpu as pltpu
from jax.experimental.pallas import tpu_sc as plsc

VOCAB = 100000
EMB = 128
B, L = 4096, 200
SCALE = math.sqrt(EMB)

NC, NS = 2, 16
NW = NC * NS
NTOK = B * L
N_PER_W = NTOK // NW
CH = 128
HCH = CH // 2
NCH = N_PER_W // CH

NBUF = 6
NPAIR = NBUF // 2
NG = NCH // NBUF
NTAIL = NCH - NG * NBUF


def _scale_chunk(rows_v, b):
    def row_pair(r, carry):
        for rr in range(2):
            for c in range(EMB // 16):
                row = b * CH + 2 * r + rr
                v = rows_v[row, pl.ds(c * 16, 16)]
                rows_v[row, pl.ds(c * 16, 16)] = v * SCALE
        return carry

    lax.fori_loop(0, CH // 2, row_pair, 0)


def _gather_body(table_hbm, idx_hbm, out_hbm, idx_v, rows_v, *sems):
    gsem, ssem = sems[: 2 * NBUF], sems[2 * NBUF :]
    wid = lax.axis_index("s") * NC + lax.axis_index("c")
    base = wid * N_PER_W

    pltpu.sync_copy(idx_hbm.at[wid], idx_v)

    def fire_gathers(j, b):
        return [
            pltpu.async_copy(
                table_hbm.at[idx_v.at[j]],
                rows_v.at[pl.ds(b * CH, CH)],
                gsem[2 * b],
            )
        ]

    def fire_pair_scatter(j, p):
        return pltpu.async_copy(
            rows_v.at[pl.ds(2 * p * CH, 2 * CH)],
            out_hbm.at[pl.ds(base + j * CH, 2 * CH)],
            ssem[p],
        )

    def drain_pair_scatter(p):
        pltpu.make_async_copy(
            rows_v.at[pl.ds(2 * p * CH, 2 * CH)],
            out_hbm.at[pl.ds(base, 2 * CH)],
            ssem[p],
        ).wait()

    def group(g, carry):
        j0 = g * NBUF
        descs = []
        for b in range(NBUF):
            if b % 2 == 0:
                p = b // 2

                @pl.when(g > 0)
                def _(p=p):
                    drain_pair_scatter(p)

            descs.append(fire_gathers(j0 + b, b))
        for b in range(NBUF):
            for d in descs[b]:
                d.wait()
            _scale_chunk(rows_v, b)
            if b % 2 == 1:
                fire_pair_scatter(j0 + b - 1, b // 2)
        return carry

    lax.fori_loop(0, NG, group, 0)

    drain_pair_scatter(0)
    tdescs = [fire_gathers(NG * NBUF + t, t) for t in range(NTAIL)]
    for t in range(NTAIL):
        for d in tdescs[t]:
            d.wait()
        _scale_chunk(rows_v, t)
    fire_pair_scatter(NG * NBUF, 0)

    for p in range(NPAIR):
        drain_pair_scatter(p)


def _sc_gather(table, idx):
    mesh = plsc.VectorSubcoreMesh(core_axis_name="c", subcore_axis_name="s")
    run = functools.partial(
        pl.kernel,
        mesh=mesh,
        out_type=jax.ShapeDtypeStruct((NTOK, EMB), jnp.float32),
        scratch_types=[
            pltpu.VMEM((NCH, CH), jnp.int32),
            pltpu.VMEM((NBUF * CH, EMB), jnp.float32),
        ]
        + [pltpu.SemaphoreType.DMA] * (2 * NBUF + NPAIR),
    )(_gather_body)
    return run(table, idx)


def kernel(tokens, table):
    idx = tokens.astype(jnp.int32).reshape(NW, NCH, CH)
    out = _sc_gather(table, idx)
    return out.reshape(B, L, EMB)


# speedup vs baseline: 1.0038x; 1.0038x over previous
import functools
import math

import jax
import jax.numpy as jnp
from jax import lax
from jax.experimental import pallas as pl
from jax.experimental.pallas import tpu as pltpu
from jax.experimental.pallas import tpu_sc as plsc

VOCAB = 100000
EMB = 128
B, L = 4096, 200
SCALE = math.sqrt(EMB)

NC, NS = 2, 16
NW = NC * NS
NTOK = B * L
N_PER_W = NTOK // NW
CH = 128
HCH = CH // 2
NCH = N_PER_W // CH

NBUF = 6
NPAIR = NBUF // 2
NG = NCH // NBUF
NTAIL = NCH - NG * NBUF


def _scale_chunk(rows_v, b):
    def row_quad(r, carry):
        for rr in range(4):
            for c in range(EMB // 16):
                row = b * CH + 4 * r + rr
                v = rows_v[row, pl.ds(c * 16, 16)]
                rows_v[row, pl.ds(c * 16, 16)] = v * SCALE
        return carry

    lax.fori_loop(0, CH // 4, row_quad, 0)


def _gather_body(table_hbm, idx_hbm, out_hbm, idx_v, rows_v, *sems):
    gsem, ssem = sems[: 2 * NBUF], sems[2 * NBUF :]
    wid = lax.axis_index("s") * NC + lax.axis_index("c")
    base = wid * N_PER_W

    pltpu.sync_copy(idx_hbm.at[wid], idx_v)

    def fire_gathers(j, b):
        descs = []
        for h in range(2):
            descs.append(
                pltpu.async_copy(
                    table_hbm.at[idx_v.at[j, pl.ds(h * HCH, HCH)]],
                    rows_v.at[pl.ds(b * CH + h * HCH, HCH)],
                    gsem[2 * b + h],
                )
            )
        return descs

    def fire_pair_scatter(j, p):
        return pltpu.async_copy(
            rows_v.at[pl.ds(2 * p * CH, 2 * CH)],
            out_hbm.at[pl.ds(base + j * CH, 2 * CH)],
            ssem[p],
        )

    def drain_pair_scatter(p):
        pltpu.make_async_copy(
            rows_v.at[pl.ds(2 * p * CH, 2 * CH)],
            out_hbm.at[pl.ds(base, 2 * CH)],
            ssem[p],
        ).wait()

    def group(g, carry):
        j0 = g * NBUF
        descs = []
        for b in range(NBUF):
            if b % 2 == 0:
                p = b // 2

                @pl.when(g > 0)
                def _(p=p):
                    drain_pair_scatter(p)

            descs.append(fire_gathers(j0 + b, b))
        for b in range(NBUF):
            for d in descs[b]:
                d.wait()
            _scale_chunk(rows_v, b)
            if b % 2 == 1:
                fire_pair_scatter(j0 + b - 1, b // 2)
        return carry

    lax.fori_loop(0, NG, group, 0)

    drain_pair_scatter(0)
    tdescs = [fire_gathers(NG * NBUF + t, t) for t in range(NTAIL)]
    for t in range(NTAIL):
        for d in tdescs[t]:
            d.wait()
        _scale_chunk(rows_v, t)
    fire_pair_scatter(NG * NBUF, 0)

    for p in range(NPAIR):
        drain_pair_scatter(p)


def _sc_gather(table, idx):
    mesh = plsc.VectorSubcoreMesh(core_axis_name="c", subcore_axis_name="s")
    run = functools.partial(
        pl.kernel,
        mesh=mesh,
        out_type=jax.ShapeDtypeStruct((NTOK, EMB), jnp.float32),
        scratch_types=[
            pltpu.VMEM((NCH, CH), jnp.int32),
            pltpu.VMEM((NBUF * CH, EMB), jnp.float32),
        ]
        + [pltpu.SemaphoreType.DMA] * (2 * NBUF + NPAIR),
    )(_gather_body)
    return run(table, idx)


def kernel(tokens, table):
    idx = tokens.astype(jnp.int32).reshape(NW, NCH, CH)
    out = _sc_gather(table, idx)
    return out.reshape(B, L, EMB)


# speedup vs baseline: 1.0049x; 1.0010x over previous
import functools
import math

import jax
import jax.numpy as jnp
from jax import lax
from jax.experimental import pallas as pl
from jax.experimental.pallas import tpu as pltpu
from jax.experimental.pallas import tpu_sc as plsc

VOCAB = 100000
EMB = 128
B, L = 4096, 200
SCALE = math.sqrt(EMB)

NC, NS = 2, 16
NW = NC * NS
NTOK = B * L
N_PER_W = NTOK // NW
CH = 128
NSPL = 4
HCH = CH // NSPL
NCH = N_PER_W // CH

NBUF = 6
NPAIR = NBUF // 2
NG = NCH // NBUF
NTAIL = NCH - NG * NBUF


def _scale_chunk(rows_v, b):
    def row_pair(r, carry):
        for rr in range(2):
            for c in range(EMB // 16):
                row = b * CH + 2 * r + rr
                v = rows_v[row, pl.ds(c * 16, 16)]
                rows_v[row, pl.ds(c * 16, 16)] = v * SCALE
        return carry

    lax.fori_loop(0, CH // 2, row_pair, 0)


def _gather_body(table_hbm, idx_hbm, out_hbm, idx_v, rows_v, *sems):
    gsem, ssem = sems[: NSPL * NBUF], sems[NSPL * NBUF :]
    wid = lax.axis_index("s") * NC + lax.axis_index("c")
    base = wid * N_PER_W

    pltpu.sync_copy(idx_hbm.at[wid], idx_v)

    def fire_gathers(j, b):
        descs = []
        for h in range(NSPL):
            descs.append(
                pltpu.async_copy(
                    table_hbm.at[idx_v.at[j, pl.ds(h * HCH, HCH)]],
                    rows_v.at[pl.ds(b * CH + h * HCH, HCH)],
                    gsem[NSPL * b + h],
                )
            )
        return descs

    def fire_pair_scatter(j, p):
        return pltpu.async_copy(
            rows_v.at[pl.ds(2 * p * CH, 2 * CH)],
            out_hbm.at[pl.ds(base + j * CH, 2 * CH)],
            ssem[p],
        )

    def drain_pair_scatter(p):
        pltpu.make_async_copy(
            rows_v.at[pl.ds(2 * p * CH, 2 * CH)],
            out_hbm.at[pl.ds(base, 2 * CH)],
            ssem[p],
        ).wait()

    def group(g, carry):
        j0 = g * NBUF
        descs = []
        for b in range(NBUF):
            if b % 2 == 0:
                p = b // 2

                @pl.when(g > 0)
                def _(p=p):
                    drain_pair_scatter(p)

            descs.append(fire_gathers(j0 + b, b))
        for b in range(NBUF):
            for d in descs[b]:
                d.wait()
            _scale_chunk(rows_v, b)
            if b % 2 == 1:
                fire_pair_scatter(j0 + b - 1, b // 2)
        return carry

    lax.fori_loop(0, NG, group, 0)

    drain_pair_scatter(0)
    tdescs = [fire_gathers(NG * NBUF + t, t) for t in range(NTAIL)]
    for t in range(NTAIL):
        for d in tdescs[t]:
            d.wait()
        _scale_chunk(rows_v, t)
    fire_pair_scatter(NG * NBUF, 0)

    for p in range(NPAIR):
        drain_pair_scatter(p)


def _sc_gather(table, idx):
    mesh = plsc.VectorSubcoreMesh(core_axis_name="c", subcore_axis_name="s")
    run = functools.partial(
        pl.kernel,
        mesh=mesh,
        out_type=jax.ShapeDtypeStruct((NTOK, EMB), jnp.float32),
        scratch_types=[
            pltpu.VMEM((NCH, CH), jnp.int32),
            pltpu.VMEM((NBUF * CH, EMB), jnp.float32),
        ]
        + [pltpu.SemaphoreType.DMA] * (NSPL * NBUF + NPAIR),
    )(_gather_body)
    return run(table, idx)


def kernel(tokens, table):
    idx = tokens.astype(jnp.int32).reshape(NW, NCH, CH)
    out = _sc_gather(table, idx)
    return out.reshape(B, L, EMB)


# speedup vs baseline: 1.0053x; 1.0004x over previous
import functools
import math

import jax
import jax.numpy as jnp
from jax import lax
from jax.experimental import pallas as pl
from jax.experimental.pallas import tpu as pltpu
from jax.experimental.pallas import tpu_sc as plsc

VOCAB = 100000
EMB = 128
B, L = 4096, 200
SCALE = math.sqrt(EMB)

NC, NS = 2, 16
NW = NC * NS
NTOK = B * L
N_PER_W = NTOK // NW
CH = 128
HCH = CH // 2
NCH = N_PER_W // CH

NBUF = 6
NPAIR = NBUF // 2
NG = NCH // NBUF
NTAIL = NCH - NG * NBUF


def _scale_chunk(rows_v, b):
    def row_pair(r, carry):
        for rr in range(2):
            for c in range(EMB // 16):
                row = b * CH + 2 * r + rr
                v = rows_v[row, pl.ds(c * 16, 16)]
                rows_v[row, pl.ds(c * 16, 16)] = v * SCALE
        return carry

    lax.fori_loop(0, CH // 2, row_pair, 0)


def _gather_body(table_hbm, idx_hbm, out_hbm, idx_v, rows_v, *sems):
    gsem, ssem = sems[: 2 * NBUF], sems[2 * NBUF :]
    wid = lax.axis_index("s") * NC + lax.axis_index("c")
    base = wid * N_PER_W

    pltpu.sync_copy(idx_hbm.at[wid], idx_v)

    def fire_gathers(j, b):
        descs = []
        for h in range(2):
            descs.append(
                pltpu.async_copy(
                    table_hbm.at[idx_v.at[j, pl.ds(h * HCH, HCH)]],
                    rows_v.at[pl.ds(b * CH + h * HCH, HCH)],
                    gsem[2 * b + h],
                )
            )
        return descs

    def fire_pair_scatter(j, p):
        return pltpu.async_copy(
            rows_v.at[pl.ds(2 * p * CH, 2 * CH)],
            out_hbm.at[pl.ds(base + j * CH, 2 * CH)],
            ssem[p],
        )

    def drain_pair_scatter(p):
        pltpu.make_async_copy(
            rows_v.at[pl.ds(2 * p * CH, 2 * CH)],
            out_hbm.at[pl.ds(base, 2 * CH)],
            ssem[p],
        ).wait()

    def group(g, carry):
        j0 = g * NBUF
        descs = []
        for b in range(NBUF):
            if b % 2 == 0:
                p = b // 2

                @pl.when(g > 0)
                def _(p=p):
                    drain_pair_scatter(p)

            descs.append(fire_gathers(j0 + b, b))
        for b in range(NBUF):
            for d in descs[b]:
                d.wait()
            _scale_chunk(rows_v, b)
            if b % 2 == 1:
                fire_pair_scatter(j0 + b - 1, b // 2)
        return carry

    lax.fori_loop(0, NG, group, 0)

    drain_pair_scatter(0)
    tdescs = [fire_gathers(NG * NBUF + t, t) for t in range(NTAIL)]
    for t in range(NTAIL):
        for d in tdescs[t]:
            d.wait()
        _scale_chunk(rows_v, t)
    fire_pair_scatter(NG * NBUF, 0)

    for p in range(NPAIR):
        drain_pair_scatter(p)


def _sc_gather(table, idx):
    mesh = plsc.VectorSubcoreMesh(core_axis_name="c", subcore_axis_name="s")
    run = functools.partial(
        pl.kernel,
        mesh=mesh,
        out_type=jax.ShapeDtypeStruct((NTOK, EMB), jnp.float32),
        scratch_types=[
            pltpu.VMEM((NCH, CH), jnp.int32),
            pltpu.VMEM((NBUF * CH, EMB), jnp.float32),
        ]
        + [pltpu.SemaphoreType.DMA] * (2 * NBUF + NPAIR),
    )(_gather_body)
    return run(table, idx)


def kernel(tokens, table):
    idx = tokens.astype(jnp.int32).reshape(NW, NCH, CH)
    out = _sc_gather(table, idx)
    return out.reshape(B, L, EMB)
